# baseline (speedup 1.0000x reference)
import functools

import jax
import jax.numpy as jnp
from jax import lax
from jax.experimental import pallas as pl
from jax.experimental.pallas import tpu as pltpu
from jax.experimental.pallas import tpu_sc as plsc

B = 16384
EMB = 16

_info = plsc.get_sparse_core_info()
_NC, _NS, _NL = _info.num_cores, _info.num_subcores, _info.num_lanes
_NW = _NC * _NS
_BPW = B // _NW
_NG = _BPW // _NL

_mesh = plsc.VectorSubcoreMesh(core_axis_name="c", subcore_axis_name="s")


def _flatten_body(chunk, in_ref, out_ref, sem):
    del sem
    for e in range(8):
        out_ref[pl.ds(e * chunk, chunk)] = in_ref[e]


def _make_flatten(n, chunk):
    npad = ((n + chunk - 1) // chunk) * chunk
    nj = npad // chunk

    def run(table_t):
        return pl.pallas_call(
            functools.partial(_flatten_body, chunk),
            grid=(2, nj),
            in_specs=[pl.BlockSpec((8, chunk), lambda ch, j: (ch, j))],
            out_specs=pl.BlockSpec((8 * chunk,), lambda ch, j: (ch * nj + j,)),
            out_shape=jax.ShapeDtypeStruct((16 * npad,), jnp.float32),
            scratch_shapes=[pltpu.SemaphoreType.DMA],
        )(table_t)

    return npad, chunk, nj, run


_UNPAD, _UCH, _UNJ, _user_flatten = _make_flatten(1000001, 262144)
_CNPAD, _CCH, _CNJ, _cat_flatten = _make_flatten(100001, 65536)
_ULOG = _UCH.bit_length() - 1
_CLOG = _CCH.bit_length() - 1


@functools.partial(
    pl.kernel,
    mesh=_mesh,
    compiler_params=pltpu.CompilerParams(
        needs_layout_passes=False, use_tc_tiling_on_sc=False),
    out_type=jax.ShapeDtypeStruct((3 * EMB, B), jnp.float32),
    scratch_types=[
        pltpu.VMEM((_BPW,), jnp.int32),
        pltpu.VMEM((_BPW,), jnp.int32),
        pltpu.VMEM((_BPW,), jnp.int32),
        pltpu.VMEM((EMB, _BPW), jnp.int32),
        pltpu.VMEM((EMB * 128,), jnp.float32),
        pltpu.VMEM((EMB * 128,), jnp.float32),
        pltpu.VMEM((3 * EMB, _BPW), jnp.float32),
        pltpu.SemaphoreType.DMA,
        pltpu.SemaphoreType.DMA,
    ],
)
def _lookup_cat_small(category_id, weekday, time_frames, cat_f, wd_p, tf_p,
                      out, cidx_v, widx_v, tidx_v, ca_v, wd_v, tf_v,
                      out_v, gsem, isem):
    wid = lax.axis_index("s") * _NC + lax.axis_index("c")
    base = wid * _BPW
    stages = [
        pltpu.async_copy(category_id.at[pl.ds(base, _BPW)], cidx_v, isem),
        pltpu.async_copy(weekday.at[pl.ds(base, _BPW)], widx_v, isem),
        pltpu.async_copy(time_frames.at[pl.ds(base, _BPW)], tidx_v, isem),
        pltpu.async_copy(wd_p, wd_v, isem),
        pltpu.async_copy(tf_p, tf_v, isem),
    ]
    for s in stages:
        s.wait()

    for g in range(_NG):
        sl = pl.ds(g * _NL, _NL)
        cvec = cidx_v[sl]
        cb = ((cvec >> _CLOG) * (8 * _CCH)) + (cvec & (_CCH - 1))
        for c in range(EMB):
            ca_v[c, sl] = cb + ((c >> 3) * 8 * _CNPAD + (c & 7) * _CCH)

    streams = [pltpu.async_copy(cat_f.at[ca_v.at[c]], out_v.at[c], gsem)
               for c in range(EMB)]

    for g in range(_NG):
        sl = pl.ds(g * _NL, _NL)
        wg = widx_v[sl]
        tg = tidx_v[sl]
        for c in range(EMB):
            out_v[EMB + c, sl] = plsc.load_gather(wd_v, [wg + c * 128])
            out_v[2 * EMB + c, sl] = plsc.load_gather(tf_v, [tg + c * 128])

    for s in streams:
        s.wait()
    pltpu.sync_copy(out_v, out.at[:, pl.ds(base, _BPW)])


@functools.partial(
    pl.kernel,
    mesh=_mesh,
    compiler_params=pltpu.CompilerParams(
        needs_layout_passes=False, use_tc_tiling_on_sc=False),
    out_type=jax.ShapeDtypeStruct((EMB, B), jnp.float32),
    scratch_types=[
        pltpu.VMEM((_BPW,), jnp.int32),
        pltpu.VMEM((EMB, _BPW), jnp.int32),
        pltpu.VMEM((EMB, _BPW), jnp.float32),
        pltpu.SemaphoreType.DMA,
    ],
)
def _lookup_user(user_id, user_f, out, uidx_v, ua_v, out_v, gsem):
    wid = lax.axis_index("s") * _NC + lax.axis_index("c")
    base = wid * _BPW
    pltpu.sync_copy(user_id.at[pl.ds(base, _BPW)], uidx_v)

    for g in range(_NG):
        sl = pl.ds(g * _NL, _NL)
        uvec = uidx_v[sl]
        ub = ((uvec >> _ULOG) * (8 * _UCH)) + (uvec & (_UCH - 1))
        for c in range(EMB):
            ua_v[c, sl] = ub + ((c >> 3) * 8 * _UNPAD + (c & 7) * _UCH)

    streams = [pltpu.async_copy(user_f.at[ua_v.at[c]], out_v.at[c], gsem)
               for c in range(EMB)]
    for s in streams:
        s.wait()
    pltpu.sync_copy(out_v, out.at[:, pl.ds(base, _BPW)])


def kernel(user_id, category_id, weekday, time_frames,
           user_table, category_table, weekday_table, time_frame_table):
    user_f = _user_flatten(user_table.T)
    cat_f = _cat_flatten(category_table.T)
    wd_p = jnp.pad(weekday_table.T, ((0, 0), (0, 128 - 10))).reshape(-1)
    tf_p = jnp.pad(time_frame_table.T, ((0, 0), (0, 128 - 26))).reshape(-1)
    rest = _lookup_cat_small(category_id, weekday, time_frames,
                             cat_f, wd_p, tf_p)
    user = _lookup_user(user_id, user_f)
    return jnp.concatenate([user.T, rest.T], axis=1)


# speedup vs baseline: 1.0356x; 1.0356x over previous
import functools

import jax
import jax.numpy as jnp
from jax import lax
from jax.experimental import pallas as pl
from jax.experimental.pallas import tpu as pltpu
from jax.experimental.pallas import tpu_sc as plsc

B = 16384
EMB = 16

_info = plsc.get_sparse_core_info()
_NC, _NS, _NL = _info.num_cores, _info.num_subcores, _info.num_lanes
_NW = _NC * _NS
_BPW = B // _NW
_NG = _BPW // _NL

_mesh = plsc.VectorSubcoreMesh(core_axis_name="c", subcore_axis_name="s")


def _flatten_body(chunk, in_ref, out_ref, sem):
    del sem
    for e in range(8):
        out_ref[pl.ds(e * chunk, chunk)] = in_ref[e]


def _make_flatten(n, chunk):
    npad = ((n + chunk - 1) // chunk) * chunk
    nj = npad // chunk

    def run(table_t):
        return pl.pallas_call(
            functools.partial(_flatten_body, chunk),
            grid=(2, nj),
            in_specs=[pl.BlockSpec((8, chunk), lambda ch, j: (ch, j))],
            out_specs=pl.BlockSpec((8 * chunk,), lambda ch, j: (ch * nj + j,)),
            out_shape=jax.ShapeDtypeStruct((16 * npad,), jnp.float32),
            scratch_shapes=[pltpu.SemaphoreType.DMA],
        )(table_t)

    return npad, chunk, nj, run


_UNPAD, _UCH, _UNJ, _user_flatten = _make_flatten(1000001, 262144)
_CNPAD, _CCH, _CNJ, _cat_flatten = _make_flatten(100001, 65536)
_ULOG = _UCH.bit_length() - 1
_CLOG = _CCH.bit_length() - 1


@functools.partial(
    pl.kernel,
    mesh=_mesh,
    compiler_params=pltpu.CompilerParams(
        needs_layout_passes=False, use_tc_tiling_on_sc=False),
    out_type=jax.ShapeDtypeStruct((4 * EMB, B), jnp.float32),
    scratch_types=[
        pltpu.VMEM((_BPW,), jnp.int32),
        pltpu.VMEM((_BPW,), jnp.int32),
        pltpu.VMEM((_BPW,), jnp.int32),
        pltpu.VMEM((_BPW,), jnp.int32),
        pltpu.VMEM((EMB, _BPW), jnp.int32),
        pltpu.VMEM((EMB, _BPW), jnp.int32),
        pltpu.VMEM((EMB * 128,), jnp.float32),
        pltpu.VMEM((EMB * 128,), jnp.float32),
        pltpu.VMEM((4 * EMB, _BPW), jnp.float32),
        pltpu.SemaphoreType.DMA,
        pltpu.SemaphoreType.DMA,
    ],
)
def _lookup_kernel(user_id, category_id, weekday, time_frames,
                   user_f, cat_f, wd_p, tf_p,
                   out, uidx_v, cidx_v, widx_v, tidx_v,
                   ua_v, ca_v, wd_v, tf_v, out_v, gsem, isem):
    wid = lax.axis_index("s") * _NC + lax.axis_index("c")
    base = wid * _BPW
    stages = [
        pltpu.async_copy(user_id.at[pl.ds(base, _BPW)], uidx_v, isem),
        pltpu.async_copy(category_id.at[pl.ds(base, _BPW)], cidx_v, isem),
        pltpu.async_copy(weekday.at[pl.ds(base, _BPW)], widx_v, isem),
        pltpu.async_copy(time_frames.at[pl.ds(base, _BPW)], tidx_v, isem),
        pltpu.async_copy(wd_p, wd_v, isem),
        pltpu.async_copy(tf_p, tf_v, isem),
    ]
    for s in stages:
        s.wait()

    for g in range(_NG):
        sl = pl.ds(g * _NL, _NL)
        uvec = uidx_v[sl]
        cvec = cidx_v[sl]
        ua_v[0, sl] = ((uvec >> _ULOG) * (8 * _UCH)) + (uvec & (_UCH - 1))
        ca_v[0, sl] = ((cvec >> _CLOG) * (8 * _CCH)) + (cvec & (_CCH - 1))

    streams = []
    for c in range(EMB - 1, -1, -1):
        ko = (c >> 3) * 8 * _UNPAD + (c & 7) * _UCH
        kc = (c >> 3) * 8 * _CNPAD + (c & 7) * _CCH
        for g in range(_NG):
            sl = pl.ds(g * _NL, _NL)
            ua_v[c, sl] = ua_v[0, sl] + ko
            ca_v[c, sl] = ca_v[0, sl] + kc
        streams.append(pltpu.async_copy(
            user_f.at[ua_v.at[c]], out_v.at[c], gsem))
        streams.append(pltpu.async_copy(
            cat_f.at[ca_v.at[c]], out_v.at[EMB + c], gsem))

    for g in range(_NG):
        sl = pl.ds(g * _NL, _NL)
        wg = widx_v[sl]
        tg = tidx_v[sl]
        for c in range(EMB):
            out_v[2 * EMB + c, sl] = plsc.load_gather(wd_v, [wg + c * 128])
            out_v[3 * EMB + c, sl] = plsc.load_gather(tf_v, [tg + c * 128])

    half = pltpu.async_copy(out_v.at[pl.ds(2 * EMB, 2 * EMB)],
                            out.at[pl.ds(2 * EMB, 2 * EMB),
                                   pl.ds(base, _BPW)], isem)
    for s in streams:
        s.wait()
    half.wait()
    pltpu.sync_copy(out_v.at[pl.ds(0, 2 * EMB)],
                    out.at[pl.ds(0, 2 * EMB), pl.ds(base, _BPW)])


def kernel(user_id, category_id, weekday, time_frames,
           user_table, category_table, weekday_table, time_frame_table):
    user_f = _user_flatten(user_table.T)
    cat_f = _cat_flatten(category_table.T)
    wd_p = jnp.pad(weekday_table.T, ((0, 0), (0, 128 - 10))).reshape(-1)
    tf_p = jnp.pad(time_frame_table.T, ((0, 0), (0, 128 - 26))).reshape(-1)
    out = _lookup_kernel(user_id, category_id, weekday, time_frames,
                         user_f, cat_f, wd_p, tf_p)
    return out.T
